# baseline (speedup 1.0000x reference)
import functools

import jax
import jax.numpy as jnp
from jax import lax
from jax.experimental import pallas as pl
from jax.experimental.pallas import tpu as pltpu
from jax.experimental.pallas import tpu_sc as plsc

N = 10000
E = 320000
P = 100000
D = 128

NPAD = 10240
PPAD = 102400
NC, NS, L = 2, 16, 16
NW = NC * NS

EPW = E // NW
ECH = 100
ENCH = EPW // ECH

PCH = 128
PPW = PPAD // NW
PNCH = PPW // PCH
PRING = 5

NB = 400
NGRID = N // NB
PB = 1024
PGRID = PPAD // PB

_mesh = plsc.VectorSubcoreMesh(core_axis_name="c", subcore_axis_name="s")


@functools.partial(
    pl.kernel,
    out_type=jax.ShapeDtypeStruct((NW, N), jnp.float32),
    mesh=_mesh,
    compiler_params=pltpu.CompilerParams(needs_layout_passes=False),
    scratch_types=[
        pltpu.VMEM((EPW,), jnp.int32),
        pltpu.VMEM((N,), jnp.float32),
    ],
)
def _deg_kernel(col_hbm, out_hbm, cidx, degv):
    w = lax.axis_index("s") * NC + lax.axis_index("c")
    pltpu.sync_copy(col_hbm.at[pl.ds(w * EPW, EPW)], cidx)
    zeros16 = jnp.zeros((L,), jnp.float32)
    ones16 = jnp.ones((L,), jnp.float32)

    def zbody(i, carry):
        degv[pl.ds(i * L, L)] = zeros16
        return carry

    lax.fori_loop(0, N // L, zbody, 0)

    def sbody(i, carry):
        idx = cidx[pl.ds(i * L, L)]
        plsc.addupdate_scatter(degv, [idx], ones16)
        return carry

    lax.fori_loop(0, EPW // L, sbody, 0)
    pltpu.sync_copy(degv, out_hbm.at[w])


@functools.partial(
    pl.kernel,
    out_type=jax.ShapeDtypeStruct((NC, NPAD, D), jnp.float32),
    mesh=_mesh,
    scratch_types=[
        [pltpu.VMEM((ECH,), jnp.int32)] * 2,
        [pltpu.VMEM((ECH,), jnp.int32)] * 2,
        [pltpu.VMEM((ECH, D), jnp.float32)] * 2,
        [pltpu.SemaphoreType.DMA] * 2,
        [pltpu.SemaphoreType.DMA] * 2,
        [pltpu.SemaphoreType.DMA] * 2,
        pltpu.VMEM_SHARED((NPAD, D), jnp.float32),
    ],
)
def _scatter_kernel(ridx_hbm, cidx_hbm, tbl_hbm, zeros_hbm, out_hbm,
                    rbuf, cbuf, rows, rsem, csem, gsem, accsh):
    c = lax.axis_index("c")
    s = lax.axis_index("s")
    w = s * NC + c

    rows_per_tile = NPAD // NS
    base = s * rows_per_tile
    pltpu.sync_copy(zeros_hbm, rows[0])
    for k in range(rows_per_tile // ECH):
        pltpu.sync_copy(rows[0], accsh.at[pl.ds(base + k * ECH, ECH)])
    rem = rows_per_tile % ECH
    pltpu.sync_copy(rows[0].at[pl.ds(0, rem)],
                    accsh.at[pl.ds(base + rows_per_tile - rem, rem)])
    plsc.subcore_barrier()

    for p in range(2):
        pltpu.sync_copy(ridx_hbm.at[w, p], rbuf[p])
        pltpu.sync_copy(cidx_hbm.at[w, p], cbuf[p])
        pltpu.async_copy(tbl_hbm.at[rbuf[p]], rows[p], gsem[p])

    def body(j, carry):
        for p in range(2):
            t = j * 2 + p
            pltpu.make_async_copy(tbl_hbm.at[rbuf[p]], rows[p], gsem[p]).wait()

            @pl.when(t + 2 < ENCH)
            def _():
                pltpu.async_copy(ridx_hbm.at[w, t + 2], rbuf[p], rsem[p])

            pltpu.sync_copy(rows[p], accsh.at[cbuf[p]], add=True)

            @pl.when(t + 2 < ENCH)
            def _():
                pltpu.async_copy(cidx_hbm.at[w, t + 2], cbuf[p], csem[p])
                pltpu.make_async_copy(ridx_hbm.at[w, t + 2], rbuf[p],
                                      rsem[p]).wait()
                pltpu.make_async_copy(cidx_hbm.at[w, t + 2], cbuf[p],
                                      csem[p]).wait()
                pltpu.async_copy(tbl_hbm.at[rbuf[p]], rows[p], gsem[p])
        return carry

    lax.fori_loop(0, ENCH // 2, body, 0)
    plsc.subcore_barrier()
    pltpu.sync_copy(accsh.at[pl.ds(base, rows_per_tile)],
                    out_hbm.at[c, pl.ds(base, rows_per_tile)])


@functools.partial(
    pl.kernel,
    out_type=jax.ShapeDtypeStruct((2, PPAD, D // 2), jnp.int32),
    mesh=_mesh,
    compiler_params=pltpu.CompilerParams(use_tc_tiling_on_sc=False),
    scratch_types=[
        pltpu.VMEM((2 * PNCH, PCH), jnp.int32),
        [pltpu.VMEM((PCH, D // 2), jnp.int32)] * PRING,
        [pltpu.SemaphoreType.DMA] * PRING,
    ],
)
def _pair_gather_kernel(z_hbm, idx_hbm, out_hbm, idx, rows, sems):
    w = lax.axis_index("s") * NC + lax.axis_index("c")
    pltpu.sync_copy(idx_hbm.at[w], idx)
    base = w * PPW

    for k in range(PRING):
        pltpu.async_copy(z_hbm.at[idx.at[k]], rows[k], sems[k])

    def body(j, carry):
        for k in range(PRING):
            t = j * PRING + k
            pltpu.make_async_copy(z_hbm.at[idx.at[t]], rows[k],
                                  sems[k]).wait()
            sel = t // PNCH
            off = base + (t % PNCH) * PCH
            pltpu.sync_copy(rows[k], out_hbm.at[sel, pl.ds(off, PCH)])

            @pl.when(t + PRING < 2 * PNCH)
            def _():
                pltpu.async_copy(z_hbm.at[idx.at[t + PRING]], rows[k],
                                 sems[k])
        return carry

    lax.fori_loop(0, 2 * PNCH // PRING, body, 0)


def _conv_pre_body(deg_ref, x_ref, w_ref, hd_ref, dinv_ref):
    dsum = jnp.sum(deg_ref[...], axis=1, keepdims=True) + 1.0
    dinv = lax.rsqrt(dsum)
    hx = jnp.dot(x_ref[...], w_ref[...], preferred_element_type=jnp.float32)
    hd_ref[...] = hx * dinv
    dinv_ref[...] = dinv


def _conv_mid_body(a_ref, hd_ref, dinv_ref, w_ref, b_ref, out_ref):
    dinv = dinv_ref[...]
    s = (a_ref[0] + a_ref[1] + hd_ref[...]) * dinv + b_ref[...]
    h = jnp.maximum(s, 0.0)
    out_ref[...] = jnp.dot(h, w_ref[...],
                           preferred_element_type=jnp.float32) * dinv


def _conv_out_body(a_ref, hd_ref, dinv_ref, b_ref, out_ref):
    z = (a_ref[0] + a_ref[1] + hd_ref[...]) * dinv_ref[...] + b_ref[...]
    out_ref[...] = z.astype(jnp.bfloat16)


def _pair_mlp_body(zs_ref, zd_ref, wa_ref, wb_ref, wc_ref, wd_ref,
                   b1_ref, w2_ref, b2_ref, out_ref):
    zs = zs_ref[0]
    zd = zd_ref[0]
    acc = jnp.dot(zs, wa_ref[...], preferred_element_type=jnp.float32)
    acc = acc + jnp.dot(zd, wb_ref[...], preferred_element_type=jnp.float32)
    acc = acc + jnp.dot(zs * zd, wc_ref[...],
                        preferred_element_type=jnp.float32)
    acc = acc + jnp.dot(jnp.abs(zs - zd), wd_ref[...],
                        preferred_element_type=jnp.float32)
    hid = jnp.maximum(acc + b1_ref[...], 0.0)
    lv = jnp.sum(hid * w2_ref[...], axis=1, keepdims=True)
    out_ref[...] = lv + b2_ref[...]


_nblk = pl.BlockSpec((NB, D), lambda i: (i, 0))
_accblk = pl.BlockSpec((2, NB, D), lambda i: (0, i, 0))
_wblk = pl.BlockSpec((D, D), lambda i: (0, 0))
_dinvblk = pl.BlockSpec((NB, 1), lambda i: (i, 0))
_biasblk = pl.BlockSpec((1, D), lambda i: (0, 0))
_scalarblk = pl.BlockSpec((1, 1), lambda i: (0, 0))

_conv_pre = pl.pallas_call(
    _conv_pre_body,
    grid=(NGRID,),
    in_specs=[pl.BlockSpec((NB, NW), lambda i: (i, 0)), _nblk, _wblk],
    out_specs=[_nblk, _dinvblk],
    out_shape=[jax.ShapeDtypeStruct((N, D), jnp.float32),
               jax.ShapeDtypeStruct((N, 1), jnp.float32)],
)

_conv_mid = pl.pallas_call(
    _conv_mid_body,
    grid=(NGRID,),
    in_specs=[_accblk, _nblk, _dinvblk, _wblk, _biasblk],
    out_specs=_nblk,
    out_shape=jax.ShapeDtypeStruct((N, D), jnp.float32),
)

_conv_out = pl.pallas_call(
    _conv_out_body,
    grid=(NGRID,),
    in_specs=[_accblk, _nblk, _dinvblk, _biasblk],
    out_specs=_nblk,
    out_shape=jax.ShapeDtypeStruct((N, D), jnp.bfloat16),
)

_zsblk = pl.BlockSpec((1, PB, D), lambda i: (0, i, 0))
_zdblk = pl.BlockSpec((1, PB, D), lambda i: (1, i, 0))
_wblk16 = pl.BlockSpec((D, D), lambda i: (0, 0))

_pair_mlp = pl.pallas_call(
    _pair_mlp_body,
    grid=(PGRID,),
    in_specs=[_zsblk, _zdblk, _wblk16, _wblk16, _wblk16, _wblk16,
              _biasblk, _biasblk, _scalarblk],
    out_specs=pl.BlockSpec((PB, 1), lambda i: (i, 0)),
    out_shape=jax.ShapeDtypeStruct((PPAD, 1), jnp.float32),
)


def kernel(x, edge_index, edge_label_index, W1, b1, W2, b2,
           Wp1, bp1, Wp2, bp2):
    f32 = jnp.float32
    row3d = edge_index[0].reshape(NW, ENCH, ECH)
    col3d = edge_index[1].reshape(NW, ENCH, ECH)
    col_flat = edge_index[1]
    eli = jnp.concatenate(
        [edge_label_index,
         jnp.zeros((2, PPAD - P), edge_label_index.dtype)], axis=1)
    pidx = jnp.concatenate([eli[0].reshape(NW, PNCH, PCH),
                            eli[1].reshape(NW, PNCH, PCH)], axis=1)
    zeros_blk = jnp.zeros((ECH, D), f32)

    deg32 = _deg_kernel(col_flat)
    hd1, dinv = _conv_pre(deg32.T, x, W1)
    acc1 = _scatter_kernel(row3d, col3d, hd1, zeros_blk)
    hd2 = _conv_mid(acc1, hd1, dinv, W2, b1.reshape(1, D))
    acc2 = _scatter_kernel(row3d, col3d, hd2, zeros_blk)
    z16 = _conv_out(acc2, hd2, dinv, b2.reshape(1, D))
    z32 = lax.bitcast_convert_type(z16.reshape(N, D // 2, 2), jnp.int32)
    zsd32 = _pair_gather_kernel(z32, pidx)
    zsd = lax.bitcast_convert_type(
        zsd32.reshape(2, PPAD, D // 2, 1), jnp.bfloat16).reshape(2, PPAD, D)

    Wp = Wp1.astype(jnp.bfloat16).reshape(4, D, D)
    logits2d = _pair_mlp(zsd, zsd, Wp[0], Wp[1], Wp[2], Wp[3],
                         bp1.reshape(1, D), Wp2.reshape(1, D),
                         bp2.reshape(1, 1))
    return logits2d.reshape(-1)[:P]


# speedup vs baseline: 1.2352x; 1.2352x over previous
import functools

import jax
import jax.numpy as jnp
from jax import lax
from jax.experimental import pallas as pl
from jax.experimental.pallas import tpu as pltpu
from jax.experimental.pallas import tpu_sc as plsc

N = 10000
E = 320000
P = 100000
D = 128

NPAD = 10240
PPAD = 102400
NC, NS, L = 2, 16, 16
NW = NC * NS

EPW = E // NW
ECH = 100
ENCH = EPW // ECH

PCH = 128
PPW = PPAD // NW
PNCH = PPW // PCH
PRING = 5

NB = 400
NGRID = N // NB
PB = 1024
PGRID = PPAD // PB

_mesh = plsc.VectorSubcoreMesh(core_axis_name="c", subcore_axis_name="s")


@functools.partial(
    pl.kernel,
    out_type=jax.ShapeDtypeStruct((NW, N), jnp.float32),
    mesh=_mesh,
    compiler_params=pltpu.CompilerParams(needs_layout_passes=False),
    scratch_types=[
        pltpu.VMEM((EPW,), jnp.int32),
        pltpu.VMEM((N,), jnp.float32),
    ],
)
def _deg_kernel(col_hbm, out_hbm, cidx, degv):
    w = lax.axis_index("s") * NC + lax.axis_index("c")
    pltpu.sync_copy(col_hbm.at[pl.ds(w * EPW, EPW)], cidx)
    zeros16 = jnp.zeros((L,), jnp.float32)
    ones16 = jnp.ones((L,), jnp.float32)

    def zbody(i, carry):
        degv[pl.ds(i * L, L)] = zeros16
        return carry

    lax.fori_loop(0, N // L, zbody, 0)

    def sbody(i, carry):
        idx = cidx[pl.ds(i * L, L)]
        plsc.addupdate_scatter(degv, [idx], ones16)
        return carry

    lax.fori_loop(0, EPW // L, sbody, 0)
    pltpu.sync_copy(degv, out_hbm.at[w])


@functools.partial(
    pl.kernel,
    out_type=jax.ShapeDtypeStruct((NC, NPAD, D), jnp.float32),
    mesh=_mesh,
    scratch_types=[
        [pltpu.VMEM((ECH,), jnp.int32)] * 2,
        [pltpu.VMEM((ECH,), jnp.int32)] * 2,
        [pltpu.VMEM((ECH, D), jnp.float32)] * 2,
        [pltpu.SemaphoreType.DMA] * 2,
        [pltpu.SemaphoreType.DMA] * 2,
        [pltpu.SemaphoreType.DMA] * 2,
        pltpu.VMEM_SHARED((NPAD, D), jnp.float32),
    ],
)
def _scatter_kernel(ridx_hbm, cidx_hbm, tbl_hbm, zeros_hbm, out_hbm,
                    rbuf, cbuf, rows, rsem, csem, gsem, accsh):
    c = lax.axis_index("c")
    s = lax.axis_index("s")
    w = s * NC + c

    rows_per_tile = NPAD // NS
    base = s * rows_per_tile
    pltpu.sync_copy(zeros_hbm, rows[0])
    for k in range(rows_per_tile // ECH):
        pltpu.sync_copy(rows[0], accsh.at[pl.ds(base + k * ECH, ECH)])
    rem = rows_per_tile % ECH
    pltpu.sync_copy(rows[0].at[pl.ds(0, rem)],
                    accsh.at[pl.ds(base + rows_per_tile - rem, rem)])
    plsc.subcore_barrier()

    for p in range(2):
        pltpu.sync_copy(ridx_hbm.at[w, p], rbuf[p])
        pltpu.sync_copy(cidx_hbm.at[w, p], cbuf[p])
        pltpu.async_copy(tbl_hbm.at[rbuf[p]], rows[p], gsem[p])

    def body(j, carry):
        for p in range(2):
            t = j * 2 + p
            pltpu.make_async_copy(tbl_hbm.at[rbuf[p]], rows[p], gsem[p]).wait()

            @pl.when(t + 2 < ENCH)
            def _():
                pltpu.async_copy(ridx_hbm.at[w, t + 2], rbuf[p], rsem[p])

            pltpu.sync_copy(rows[p], accsh.at[cbuf[p]], add=True)

            @pl.when(t + 2 < ENCH)
            def _():
                pltpu.async_copy(cidx_hbm.at[w, t + 2], cbuf[p], csem[p])
                pltpu.make_async_copy(ridx_hbm.at[w, t + 2], rbuf[p],
                                      rsem[p]).wait()
                pltpu.make_async_copy(cidx_hbm.at[w, t + 2], cbuf[p],
                                      csem[p]).wait()
                pltpu.async_copy(tbl_hbm.at[rbuf[p]], rows[p], gsem[p])
        return carry

    lax.fori_loop(0, ENCH // 2, body, 0)
    plsc.subcore_barrier()
    pltpu.sync_copy(accsh.at[pl.ds(base, rows_per_tile)],
                    out_hbm.at[c, pl.ds(base, rows_per_tile)])


@functools.partial(
    pl.kernel,
    out_type=jax.ShapeDtypeStruct((2, PPAD, D), jnp.float32),
    mesh=_mesh,
    scratch_types=[
        pltpu.VMEM((2 * PNCH, PCH), jnp.int32),
        [pltpu.VMEM((PCH, D), jnp.float32)] * PRING,
        [pltpu.SemaphoreType.DMA] * PRING,
        [pltpu.SemaphoreType.DMA] * PRING,
    ],
)
def _pair_gather_kernel(z_hbm, idx_hbm, out_hbm, idx, rows, gsems, wsems):
    w = lax.axis_index("s") * NC + lax.axis_index("c")
    pltpu.sync_copy(idx_hbm.at[w], idx)
    base = w * PPW
    NCHT = 2 * PNCH

    def _write_dst(t):
        sel = t // PNCH
        off = base + (t % PNCH) * PCH
        return out_hbm.at[sel, pl.ds(off, PCH)]

    for k in range(2):
        pltpu.async_copy(z_hbm.at[idx.at[k]], rows[k], gsems[k])

    def body(j, carry):
        for k in range(PRING):
            v = j * PRING + k
            pltpu.make_async_copy(z_hbm.at[idx.at[v]], rows[k],
                                  gsems[k]).wait()
            pltpu.async_copy(rows[k], _write_dst(v), wsems[k])
            u = v + 2
            q = (k + 2) % PRING

            @pl.when(u < NCHT)
            def _():
                @pl.when(u >= PRING)
                def _():
                    pltpu.make_async_copy(rows[q], _write_dst(u - PRING),
                                          wsems[q]).wait()

                pltpu.async_copy(z_hbm.at[idx.at[u]], rows[q], gsems[q])
        return carry

    lax.fori_loop(0, NCHT // PRING, body, 0)
    for k in range(PRING):
        pltpu.make_async_copy(rows[k], _write_dst(NCHT - PRING + k),
                              wsems[k]).wait()


def _conv_pre_body(deg_ref, x_ref, w_ref, hd_ref, dinv_ref):
    dsum = jnp.sum(deg_ref[...], axis=1, keepdims=True) + 1.0
    dinv = lax.rsqrt(dsum)
    hx = jnp.dot(x_ref[...], w_ref[...], preferred_element_type=jnp.float32)
    hd_ref[...] = hx * dinv
    dinv_ref[...] = dinv


def _conv_mid_body(a_ref, hd_ref, dinv_ref, w_ref, b_ref, out_ref):
    dinv = dinv_ref[...]
    s = (a_ref[0] + a_ref[1] + hd_ref[...]) * dinv + b_ref[...]
    h = jnp.maximum(s, 0.0)
    out_ref[...] = jnp.dot(h, w_ref[...],
                           preferred_element_type=jnp.float32) * dinv


def _conv_out_body(a_ref, hd_ref, dinv_ref, b_ref, out_ref):
    out_ref[...] = (a_ref[0] + a_ref[1] + hd_ref[...]) * dinv_ref[...] \
        + b_ref[...]


def _pair_mlp_body(zs_ref, zd_ref, wa_ref, wb_ref, wc_ref, wd_ref,
                   b1_ref, w2_ref, b2_ref, out_ref):
    zs = zs_ref[0]
    zd = zd_ref[0]
    acc = jnp.dot(zs, wa_ref[...], preferred_element_type=jnp.float32)
    acc = acc + jnp.dot(zd, wb_ref[...], preferred_element_type=jnp.float32)
    acc = acc + jnp.dot(zs * zd, wc_ref[...],
                        preferred_element_type=jnp.float32)
    acc = acc + jnp.dot(jnp.abs(zs - zd), wd_ref[...],
                        preferred_element_type=jnp.float32)
    hid = jnp.maximum(acc + b1_ref[...], 0.0)
    lv = jnp.sum(hid * w2_ref[...], axis=1, keepdims=True)
    out_ref[...] = lv + b2_ref[...]


_nblk = pl.BlockSpec((NB, D), lambda i: (i, 0))
_accblk = pl.BlockSpec((2, NB, D), lambda i: (0, i, 0))
_wblk = pl.BlockSpec((D, D), lambda i: (0, 0))
_dinvblk = pl.BlockSpec((NB, 1), lambda i: (i, 0))
_biasblk = pl.BlockSpec((1, D), lambda i: (0, 0))
_scalarblk = pl.BlockSpec((1, 1), lambda i: (0, 0))

_conv_pre = pl.pallas_call(
    _conv_pre_body,
    grid=(NGRID,),
    in_specs=[pl.BlockSpec((NB, NW), lambda i: (i, 0)), _nblk, _wblk],
    out_specs=[_nblk, _dinvblk],
    out_shape=[jax.ShapeDtypeStruct((N, D), jnp.float32),
               jax.ShapeDtypeStruct((N, 1), jnp.float32)],
)

_conv_mid = pl.pallas_call(
    _conv_mid_body,
    grid=(NGRID,),
    in_specs=[_accblk, _nblk, _dinvblk, _wblk, _biasblk],
    out_specs=_nblk,
    out_shape=jax.ShapeDtypeStruct((N, D), jnp.float32),
)

_conv_out = pl.pallas_call(
    _conv_out_body,
    grid=(NGRID,),
    in_specs=[_accblk, _nblk, _dinvblk, _biasblk],
    out_specs=_nblk,
    out_shape=jax.ShapeDtypeStruct((N, D), jnp.float32),
)

_zsblk = pl.BlockSpec((1, PB, D), lambda i: (0, i, 0))
_zdblk = pl.BlockSpec((1, PB, D), lambda i: (1, i, 0))
_wblk16 = pl.BlockSpec((D, D), lambda i: (0, 0))

_pair_mlp = pl.pallas_call(
    _pair_mlp_body,
    grid=(PGRID,),
    in_specs=[_zsblk, _zdblk, _wblk16, _wblk16, _wblk16, _wblk16,
              _biasblk, _biasblk, _scalarblk],
    out_specs=pl.BlockSpec((PB, 1), lambda i: (i, 0)),
    out_shape=jax.ShapeDtypeStruct((PPAD, 1), jnp.float32),
)


def kernel(x, edge_index, edge_label_index, W1, b1, W2, b2,
           Wp1, bp1, Wp2, bp2):
    f32 = jnp.float32
    row3d = edge_index[0].reshape(NW, ENCH, ECH)
    col3d = edge_index[1].reshape(NW, ENCH, ECH)
    col_flat = edge_index[1]
    eli = jnp.concatenate(
        [edge_label_index,
         jnp.zeros((2, PPAD - P), edge_label_index.dtype)], axis=1)
    pidx = jnp.concatenate([eli[0].reshape(NW, PNCH, PCH),
                            eli[1].reshape(NW, PNCH, PCH)], axis=1)
    zeros_blk = jnp.zeros((ECH, D), f32)

    deg32 = _deg_kernel(col_flat)
    hd1, dinv = _conv_pre(deg32.T, x, W1)
    acc1 = _scatter_kernel(row3d, col3d, hd1, zeros_blk)
    hd2 = _conv_mid(acc1, hd1, dinv, W2, b1.reshape(1, D))
    acc2 = _scatter_kernel(row3d, col3d, hd2, zeros_blk)
    z16 = _conv_out(acc2, hd2, dinv, b2.reshape(1, D))
    zsd = _pair_gather_kernel(z16, pidx)

    Wp = Wp1.reshape(4, D, D)
    logits2d = _pair_mlp(zsd, zsd, Wp[0], Wp[1], Wp[2], Wp[3],
                         bp1.reshape(1, D), Wp2.reshape(1, D),
                         bp2.reshape(1, 1))
    return logits2d.reshape(-1)[:P]


# speedup vs baseline: 1.3063x; 1.0575x over previous
import functools

import jax
import jax.numpy as jnp
from jax import lax
from jax.experimental import pallas as pl
from jax.experimental.pallas import tpu as pltpu
from jax.experimental.pallas import tpu_sc as plsc

N = 10000
E = 320000
P = 100000
D = 128

NPAD = 10240
PPAD = 102400
NC, NS, L = 2, 16, 16
NW = NC * NS

EPW = E // NW
ECH = 100
ENCH = EPW // ECH

PCH = 128
PPW = PPAD // NW
PNCH = PPW // PCH
PRING = 5

NB = 400
NGRID = N // NB
PB = 1024
PGRID = PPAD // PB

_mesh = plsc.VectorSubcoreMesh(core_axis_name="c", subcore_axis_name="s")


@functools.partial(
    pl.kernel,
    out_type=jax.ShapeDtypeStruct((NW, N), jnp.float32),
    mesh=_mesh,
    compiler_params=pltpu.CompilerParams(needs_layout_passes=False),
    scratch_types=[
        pltpu.VMEM((EPW,), jnp.int32),
        pltpu.VMEM((N,), jnp.float32),
    ],
)
def _deg_kernel(col_hbm, out_hbm, cidx, degv):
    w = lax.axis_index("s") * NC + lax.axis_index("c")
    pltpu.sync_copy(col_hbm.at[pl.ds(w * EPW, EPW)], cidx)
    zeros16 = jnp.zeros((L,), jnp.float32)
    ones16 = jnp.ones((L,), jnp.float32)

    def zbody(i, carry):
        degv[pl.ds(i * L, L)] = zeros16
        return carry

    lax.fori_loop(0, N // L, zbody, 0)

    def sbody(i, carry):
        idx = cidx[pl.ds(i * L, L)]
        plsc.addupdate_scatter(degv, [idx], ones16)
        return carry

    lax.fori_loop(0, EPW // L, sbody, 0)
    pltpu.sync_copy(degv, out_hbm.at[w])


@functools.partial(
    pl.kernel,
    out_type=jax.ShapeDtypeStruct((NC, NPAD, D), jnp.float32),
    mesh=_mesh,
    scratch_types=[
        [pltpu.VMEM((ECH,), jnp.int32)] * 2,
        [pltpu.VMEM((ECH,), jnp.int32)] * 2,
        [pltpu.VMEM((ECH, D), jnp.float32)] * 2,
        [pltpu.SemaphoreType.DMA] * 2,
        [pltpu.SemaphoreType.DMA] * 2,
        [pltpu.SemaphoreType.DMA] * 2,
        pltpu.VMEM_SHARED((NPAD, D), jnp.float32),
    ],
)
def _scatter_kernel(ridx_hbm, cidx_hbm, tbl_hbm, zeros_hbm, out_hbm,
                    rbuf, cbuf, rows, rsem, csem, gsem, accsh):
    c = lax.axis_index("c")
    s = lax.axis_index("s")
    w = s * NC + c

    rows_per_tile = NPAD // NS
    base = s * rows_per_tile
    pltpu.sync_copy(zeros_hbm, rows[0])
    for k in range(rows_per_tile // ECH):
        pltpu.sync_copy(rows[0], accsh.at[pl.ds(base + k * ECH, ECH)])
    rem = rows_per_tile % ECH
    pltpu.sync_copy(rows[0].at[pl.ds(0, rem)],
                    accsh.at[pl.ds(base + rows_per_tile - rem, rem)])
    plsc.subcore_barrier()

    for p in range(2):
        pltpu.sync_copy(ridx_hbm.at[w, p], rbuf[p])
        pltpu.sync_copy(cidx_hbm.at[w, p], cbuf[p])
        pltpu.async_copy(tbl_hbm.at[rbuf[p]], rows[p], gsem[p])

    def body(j, carry):
        for p in range(2):
            t = j * 2 + p
            pltpu.make_async_copy(tbl_hbm.at[rbuf[p]], rows[p], gsem[p]).wait()

            @pl.when(t + 2 < ENCH)
            def _():
                pltpu.async_copy(ridx_hbm.at[w, t + 2], rbuf[p], rsem[p])

            pltpu.sync_copy(rows[p], accsh.at[cbuf[p]], add=True)

            @pl.when(t + 2 < ENCH)
            def _():
                pltpu.async_copy(cidx_hbm.at[w, t + 2], cbuf[p], csem[p])
                pltpu.make_async_copy(ridx_hbm.at[w, t + 2], rbuf[p],
                                      rsem[p]).wait()
                pltpu.make_async_copy(cidx_hbm.at[w, t + 2], cbuf[p],
                                      csem[p]).wait()
                pltpu.async_copy(tbl_hbm.at[rbuf[p]], rows[p], gsem[p])
        return carry

    lax.fori_loop(0, ENCH // 2, body, 0)
    plsc.subcore_barrier()
    pltpu.sync_copy(accsh.at[pl.ds(base, rows_per_tile)],
                    out_hbm.at[c, pl.ds(base, rows_per_tile)])


NSL = 5
SNCH = PNCH // NSL
SPW = SNCH * PCH
SP = NW * SPW


@functools.partial(
    pl.kernel,
    out_type=jax.ShapeDtypeStruct((2, SP, D), jnp.float32),
    mesh=_mesh,
    scratch_types=[
        pltpu.VMEM((2 * SNCH, PCH), jnp.int32),
        [pltpu.VMEM((PCH, D), jnp.float32)] * PRING,
        [pltpu.SemaphoreType.DMA] * PRING,
    ],
)
def _pair_gather_kernel(z_hbm, idx_hbm, out_hbm, idx, rows, sems):
    w = lax.axis_index("s") * NC + lax.axis_index("c")
    pltpu.sync_copy(idx_hbm.at[w], idx)
    base = w * SPW
    NCHT = 2 * SNCH

    for k in range(PRING):
        pltpu.async_copy(z_hbm.at[idx.at[k]], rows[k], sems[k])

    def body(j, carry):
        for k in range(PRING):
            t = j * PRING + k
            pltpu.make_async_copy(z_hbm.at[idx.at[t]], rows[k],
                                  sems[k]).wait()
            sel = t // SNCH
            off = base + (t % SNCH) * PCH
            pltpu.sync_copy(rows[k], out_hbm.at[sel, pl.ds(off, PCH)])

            @pl.when(t + PRING < NCHT)
            def _():
                pltpu.async_copy(z_hbm.at[idx.at[t + PRING]], rows[k],
                                 sems[k])
        return carry

    lax.fori_loop(0, NCHT // PRING, body, 0)


def _conv_pre_body(deg_ref, x_ref, w_ref, hd_ref, dinv_ref):
    dsum = jnp.sum(deg_ref[...], axis=1, keepdims=True) + 1.0
    dinv = lax.rsqrt(dsum)
    hx = jnp.dot(x_ref[...], w_ref[...], preferred_element_type=jnp.float32)
    hd_ref[...] = hx * dinv
    dinv_ref[...] = dinv


def _conv_mid_body(a_ref, hd_ref, dinv_ref, w_ref, b_ref, out_ref):
    dinv = dinv_ref[...]
    s = (a_ref[0] + a_ref[1] + hd_ref[...]) * dinv + b_ref[...]
    h = jnp.maximum(s, 0.0)
    out_ref[...] = jnp.dot(h, w_ref[...],
                           preferred_element_type=jnp.float32) * dinv


def _conv_out_body(a_ref, hd_ref, dinv_ref, b_ref, out_ref):
    out_ref[...] = (a_ref[0] + a_ref[1] + hd_ref[...]) * dinv_ref[...] \
        + b_ref[...]


def _pair_mlp_body(zs_ref, zd_ref, wa_ref, wb_ref, wc_ref, wd_ref,
                   b1_ref, w2_ref, b2_ref, out_ref):
    zs = zs_ref[0]
    zd = zd_ref[0]
    acc = jnp.dot(zs, wa_ref[...], preferred_element_type=jnp.float32)
    acc = acc + jnp.dot(zd, wb_ref[...], preferred_element_type=jnp.float32)
    acc = acc + jnp.dot(zs * zd, wc_ref[...],
                        preferred_element_type=jnp.float32)
    acc = acc + jnp.dot(jnp.abs(zs - zd), wd_ref[...],
                        preferred_element_type=jnp.float32)
    hid = jnp.maximum(acc + b1_ref[...], 0.0)
    lv = jnp.sum(hid * w2_ref[...], axis=1, keepdims=True)
    out_ref[...] = lv + b2_ref[...]


_nblk = pl.BlockSpec((NB, D), lambda i: (i, 0))
_accblk = pl.BlockSpec((2, NB, D), lambda i: (0, i, 0))
_wblk = pl.BlockSpec((D, D), lambda i: (0, 0))
_dinvblk = pl.BlockSpec((NB, 1), lambda i: (i, 0))
_biasblk = pl.BlockSpec((1, D), lambda i: (0, 0))
_scalarblk = pl.BlockSpec((1, 1), lambda i: (0, 0))

_conv_pre = pl.pallas_call(
    _conv_pre_body,
    grid=(NGRID,),
    in_specs=[pl.BlockSpec((NB, NW), lambda i: (i, 0)), _nblk, _wblk],
    out_specs=[_nblk, _dinvblk],
    out_shape=[jax.ShapeDtypeStruct((N, D), jnp.float32),
               jax.ShapeDtypeStruct((N, 1), jnp.float32)],
)

_conv_mid = pl.pallas_call(
    _conv_mid_body,
    grid=(NGRID,),
    in_specs=[_accblk, _nblk, _dinvblk, _wblk, _biasblk],
    out_specs=_nblk,
    out_shape=jax.ShapeDtypeStruct((N, D), jnp.float32),
)

_conv_out = pl.pallas_call(
    _conv_out_body,
    grid=(NGRID,),
    in_specs=[_accblk, _nblk, _dinvblk, _biasblk],
    out_specs=_nblk,
    out_shape=jax.ShapeDtypeStruct((N, D), jnp.float32),
)

_zsblk = pl.BlockSpec((1, PB, D), lambda i: (0, i, 0))
_zdblk = pl.BlockSpec((1, PB, D), lambda i: (1, i, 0))
_wblk16 = pl.BlockSpec((D, D), lambda i: (0, 0))

_pair_mlp = pl.pallas_call(
    _pair_mlp_body,
    grid=(SP // PB,),
    in_specs=[_zsblk, _zdblk, _wblk16, _wblk16, _wblk16, _wblk16,
              _biasblk, _biasblk, _scalarblk],
    out_specs=pl.BlockSpec((PB, 1), lambda i: (i, 0)),
    out_shape=jax.ShapeDtypeStruct((SP, 1), jnp.float32),
)


def kernel(x, edge_index, edge_label_index, W1, b1, W2, b2,
           Wp1, bp1, Wp2, bp2):
    f32 = jnp.float32
    row3d = edge_index[0].reshape(NW, ENCH, ECH)
    col3d = edge_index[1].reshape(NW, ENCH, ECH)
    col_flat = edge_index[1]
    eli = jnp.concatenate(
        [edge_label_index,
         jnp.zeros((2, PPAD - P), edge_label_index.dtype)], axis=1)
    pidx = jnp.concatenate([eli[0].reshape(NW, PNCH, PCH),
                            eli[1].reshape(NW, PNCH, PCH)], axis=1)
    zeros_blk = jnp.zeros((ECH, D), f32)

    deg32 = _deg_kernel(col_flat)
    hd1, dinv = _conv_pre(deg32.T, x, W1)
    acc1 = _scatter_kernel(row3d, col3d, hd1, zeros_blk)
    hd2 = _conv_mid(acc1, hd1, dinv, W2, b1.reshape(1, D))
    acc2 = _scatter_kernel(row3d, col3d, hd2, zeros_blk)
    z = _conv_out(acc2, hd2, dinv, b2.reshape(1, D))

    Wp = Wp1.reshape(4, D, D)
    bp1r = bp1.reshape(1, D)
    w2r = Wp2.reshape(1, D)
    bp2r = bp2.reshape(1, 1)
    parts = []
    for sl in range(NSL):
        pidx_s = jnp.concatenate(
            [pidx[:, sl * SNCH:(sl + 1) * SNCH],
             pidx[:, PNCH + sl * SNCH:PNCH + (sl + 1) * SNCH]], axis=1)
        zsd_s = _pair_gather_kernel(z, pidx_s)
        parts.append(_pair_mlp(zsd_s, zsd_s, Wp[0], Wp[1], Wp[2], Wp[3],
                               bp1r, w2r, bp2r))
    logits = jnp.stack(parts).reshape(NSL, NW, SPW).transpose(1, 0, 2)
    return logits.reshape(-1)[:P]


# speedup vs baseline: 1.3596x; 1.0408x over previous
import functools

import jax
import jax.numpy as jnp
from jax import lax
from jax.experimental import pallas as pl
from jax.experimental.pallas import tpu as pltpu
from jax.experimental.pallas import tpu_sc as plsc

N = 10000
E = 320000
P = 100000
D = 128

NPAD = 10240
PPAD = 102400
NC, NS, L = 2, 16, 16
NW = NC * NS

EPW = E // NW
ECH = 100
ENCH = EPW // ECH

PCH = 128
PPW = PPAD // NW
PNCH = PPW // PCH
PRING = 5

NB = 400
NGRID = N // NB
PB = 1024
PGRID = PPAD // PB

_mesh = plsc.VectorSubcoreMesh(core_axis_name="c", subcore_axis_name="s")


@functools.partial(
    pl.kernel,
    out_type=jax.ShapeDtypeStruct((NW, N), jnp.float32),
    mesh=_mesh,
    compiler_params=pltpu.CompilerParams(needs_layout_passes=False),
    scratch_types=[
        pltpu.VMEM((EPW,), jnp.int32),
        pltpu.VMEM((N,), jnp.float32),
    ],
)
def _deg_kernel(col_hbm, out_hbm, cidx, degv):
    w = lax.axis_index("s") * NC + lax.axis_index("c")
    pltpu.sync_copy(col_hbm.at[pl.ds(w * EPW, EPW)], cidx)
    zeros16 = jnp.zeros((L,), jnp.float32)
    ones16 = jnp.ones((L,), jnp.float32)

    def zbody(i, carry):
        degv[pl.ds(i * L, L)] = zeros16
        return carry

    lax.fori_loop(0, N // L, zbody, 0)

    def sbody(i, carry):
        idx = cidx[pl.ds(i * L, L)]
        plsc.addupdate_scatter(degv, [idx], ones16)
        return carry

    lax.fori_loop(0, EPW // L, sbody, 0)
    pltpu.sync_copy(degv, out_hbm.at[w])


@functools.partial(
    pl.kernel,
    out_type=jax.ShapeDtypeStruct((NC, NPAD, D), jnp.float32),
    mesh=_mesh,
    scratch_types=[
        [pltpu.VMEM((ECH,), jnp.int32)] * 2,
        [pltpu.VMEM((ECH,), jnp.int32)] * 2,
        [pltpu.VMEM((ECH, D), jnp.float32)] * 2,
        [pltpu.SemaphoreType.DMA] * 2,
        [pltpu.SemaphoreType.DMA] * 2,
        [pltpu.SemaphoreType.DMA] * 2,
        pltpu.VMEM_SHARED((NPAD, D), jnp.float32),
    ],
)
def _scatter_kernel(ridx_hbm, cidx_hbm, tbl_hbm, zeros_hbm, out_hbm,
                    rbuf, cbuf, rows, rsem, csem, gsem, accsh):
    c = lax.axis_index("c")
    s = lax.axis_index("s")
    w = s * NC + c

    rows_per_tile = NPAD // NS
    base = s * rows_per_tile
    pltpu.sync_copy(zeros_hbm, rows[0])
    for k in range(rows_per_tile // ECH):
        pltpu.sync_copy(rows[0], accsh.at[pl.ds(base + k * ECH, ECH)])
    rem = rows_per_tile % ECH
    pltpu.sync_copy(rows[0].at[pl.ds(0, rem)],
                    accsh.at[pl.ds(base + rows_per_tile - rem, rem)])
    plsc.subcore_barrier()

    for p in range(2):
        pltpu.sync_copy(ridx_hbm.at[w, p], rbuf[p])
        pltpu.sync_copy(cidx_hbm.at[w, p], cbuf[p])
        pltpu.async_copy(tbl_hbm.at[rbuf[p]], rows[p], gsem[p])

    def body(j, carry):
        for p in range(2):
            t = j * 2 + p
            pltpu.make_async_copy(tbl_hbm.at[rbuf[p]], rows[p], gsem[p]).wait()

            @pl.when(t + 2 < ENCH)
            def _():
                pltpu.async_copy(ridx_hbm.at[w, t + 2], rbuf[p], rsem[p])

            pltpu.sync_copy(rows[p], accsh.at[cbuf[p]], add=True)

            @pl.when(t + 2 < ENCH)
            def _():
                pltpu.async_copy(cidx_hbm.at[w, t + 2], cbuf[p], csem[p])
                pltpu.make_async_copy(ridx_hbm.at[w, t + 2], rbuf[p],
                                      rsem[p]).wait()
                pltpu.make_async_copy(cidx_hbm.at[w, t + 2], cbuf[p],
                                      csem[p]).wait()
                pltpu.async_copy(tbl_hbm.at[rbuf[p]], rows[p], gsem[p])
        return carry

    lax.fori_loop(0, ENCH // 2, body, 0)
    plsc.subcore_barrier()
    pltpu.sync_copy(accsh.at[pl.ds(base, rows_per_tile)],
                    out_hbm.at[c, pl.ds(base, rows_per_tile)])


NSL = 5
SP = PPAD // NSL
SCH0 = 2
SCH1 = 8
PRING = 4
SMAX = SCH0 + SCH1


@functools.partial(
    pl.kernel,
    out_type=jax.ShapeDtypeStruct((2, SP, D), jnp.float32),
    mesh=_mesh,
    scratch_types=[
        pltpu.VMEM((2 * SMAX, PCH), jnp.int32),
        [pltpu.VMEM((PCH, D), jnp.float32)] * PRING,
        [pltpu.SemaphoreType.DMA] * PRING,
    ],
)
def _pair_gather_kernel(z_hbm, idx_hbm, out_hbm, idx, rows, sems):
    c = lax.axis_index("c")
    s = lax.axis_index("s")
    w = s * NC + c
    pltpu.sync_copy(idx_hbm.at[w], idx)
    nch = jnp.where(c == 0, SCH0, SCH1)
    jobs = 2 * nch
    base_pair = jnp.where(c == 0, s * (SCH0 * PCH),
                          16 * (SCH0 * PCH) + s * (SCH1 * PCH))

    def _gather(t, k):
        side = t // nch
        jj = t - side * nch
        pltpu.async_copy(z_hbm.at[idx.at[side * SMAX + jj]], rows[k],
                         sems[k])

    def _drain(t, k):
        side = t // nch
        jj = t - side * nch
        pltpu.make_async_copy(z_hbm.at[idx.at[side * SMAX + jj]], rows[k],
                              sems[k]).wait()
        pltpu.sync_copy(rows[k],
                        out_hbm.at[side, pl.ds(base_pair + jj * PCH, PCH)])

    for k in range(PRING):
        _gather(k, k)

    def body(j, carry):
        for k in range(PRING):
            t = j * PRING + k
            _drain(t, k)

            @pl.when(t + PRING < jobs)
            def _():
                _gather(t + PRING, k)
        return carry

    lax.fori_loop(0, jobs // PRING, body, 0)


def _conv_pre_body(deg_ref, x_ref, w_ref, hd_ref, dinv_ref):
    dsum = jnp.sum(deg_ref[...], axis=1, keepdims=True) + 1.0
    dinv = lax.rsqrt(dsum)
    hx = jnp.dot(x_ref[...], w_ref[...], preferred_element_type=jnp.float32)
    hd_ref[...] = hx * dinv
    dinv_ref[...] = dinv


def _conv_mid_body(a_ref, hd_ref, dinv_ref, w_ref, b_ref, out_ref):
    dinv = dinv_ref[...]
    s = (a_ref[0] + a_ref[1] + hd_ref[...]) * dinv + b_ref[...]
    h = jnp.maximum(s, 0.0)
    out_ref[...] = jnp.dot(h, w_ref[...],
                           preferred_element_type=jnp.float32) * dinv


def _conv_out_body(a_ref, hd_ref, dinv_ref, b_ref, out_ref):
    out_ref[...] = (a_ref[0] + a_ref[1] + hd_ref[...]) * dinv_ref[...] \
        + b_ref[...]


def _pair_mlp_body(zs_ref, zd_ref, wa_ref, wb_ref, wc_ref, wd_ref,
                   b1_ref, w2_ref, b2_ref, out_ref):
    zs = zs_ref[0]
    zd = zd_ref[0]
    acc = jnp.dot(zs, wa_ref[...], preferred_element_type=jnp.float32)
    acc = acc + jnp.dot(zd, wb_ref[...], preferred_element_type=jnp.float32)
    acc = acc + jnp.dot(zs * zd, wc_ref[...],
                        preferred_element_type=jnp.float32)
    acc = acc + jnp.dot(jnp.abs(zs - zd), wd_ref[...],
                        preferred_element_type=jnp.float32)
    hid = jnp.maximum(acc + b1_ref[...], 0.0)
    lv = jnp.sum(hid * w2_ref[...], axis=1, keepdims=True)
    out_ref[...] = lv + b2_ref[...]


_nblk = pl.BlockSpec((NB, D), lambda i: (i, 0))
_accblk = pl.BlockSpec((2, NB, D), lambda i: (0, i, 0))
_wblk = pl.BlockSpec((D, D), lambda i: (0, 0))
_dinvblk = pl.BlockSpec((NB, 1), lambda i: (i, 0))
_biasblk = pl.BlockSpec((1, D), lambda i: (0, 0))
_scalarblk = pl.BlockSpec((1, 1), lambda i: (0, 0))

_conv_pre = pl.pallas_call(
    _conv_pre_body,
    grid=(NGRID,),
    in_specs=[pl.BlockSpec((NB, NW), lambda i: (i, 0)), _nblk, _wblk],
    out_specs=[_nblk, _dinvblk],
    out_shape=[jax.ShapeDtypeStruct((N, D), jnp.float32),
               jax.ShapeDtypeStruct((N, 1), jnp.float32)],
)

_conv_mid = pl.pallas_call(
    _conv_mid_body,
    grid=(NGRID,),
    in_specs=[_accblk, _nblk, _dinvblk, _wblk, _biasblk],
    out_specs=_nblk,
    out_shape=jax.ShapeDtypeStruct((N, D), jnp.float32),
)

_conv_out = pl.pallas_call(
    _conv_out_body,
    grid=(NGRID,),
    in_specs=[_accblk, _nblk, _dinvblk, _biasblk],
    out_specs=_nblk,
    out_shape=jax.ShapeDtypeStruct((N, D), jnp.float32),
)

_zsblk = pl.BlockSpec((1, PB, D), lambda i: (0, i, 0))
_zdblk = pl.BlockSpec((1, PB, D), lambda i: (1, i, 0))
_wblk16 = pl.BlockSpec((D, D), lambda i: (0, 0))

_pair_mlp = pl.pallas_call(
    _pair_mlp_body,
    grid=(SP // PB,),
    in_specs=[_zsblk, _zdblk, _wblk16, _wblk16, _wblk16, _wblk16,
              _biasblk, _biasblk, _scalarblk],
    out_specs=pl.BlockSpec((PB, 1), lambda i: (i, 0)),
    out_shape=jax.ShapeDtypeStruct((SP, 1), jnp.float32),
)


def kernel(x, edge_index, edge_label_index, W1, b1, W2, b2,
           Wp1, bp1, Wp2, bp2):
    f32 = jnp.float32
    row3d = edge_index[0].reshape(NW, ENCH, ECH)
    col3d = edge_index[1].reshape(NW, ENCH, ECH)
    col_flat = edge_index[1]
    eli = jnp.concatenate(
        [edge_label_index,
         jnp.zeros((2, PPAD - P), edge_label_index.dtype)], axis=1)
    zeros_blk = jnp.zeros((ECH, D), f32)

    deg32 = _deg_kernel(col_flat)
    hd1, dinv = _conv_pre(deg32.T, x, W1)
    acc1 = _scatter_kernel(row3d, col3d, hd1, zeros_blk)
    hd2 = _conv_mid(acc1, hd1, dinv, W2, b1.reshape(1, D))
    acc2 = _scatter_kernel(row3d, col3d, hd2, zeros_blk)
    z = _conv_out(acc2, hd2, dinv, b2.reshape(1, D))

    Wp = Wp1.reshape(4, D, D)
    bp1r = bp1.reshape(1, D)
    w2r = Wp2.reshape(1, D)
    bp2r = bp2.reshape(1, 1)

    def _slice_idx(v):
        c0 = v[:16 * SCH0 * PCH].reshape(16, SCH0, PCH)
        c0 = jnp.concatenate(
            [c0, jnp.zeros((16, SMAX - SCH0, PCH), v.dtype)], axis=1)
        c1 = v[16 * SCH0 * PCH:].reshape(16, SCH1, PCH)
        c1 = jnp.concatenate(
            [c1, jnp.zeros((16, SMAX - SCH1, PCH), v.dtype)], axis=1)
        return jnp.stack([c0, c1], axis=1).reshape(NW, SMAX, PCH)

    parts = []
    for sl in range(NSL):
        lo = sl * SP
        pidx_s = jnp.concatenate([_slice_idx(eli[0][lo:lo + SP]),
                                  _slice_idx(eli[1][lo:lo + SP])], axis=1)
        zsd_s = _pair_gather_kernel(z, pidx_s)
        parts.append(_pair_mlp(zsd_s, zsd_s, Wp[0], Wp[1], Wp[2], Wp[3],
                               bp1r, w2r, bp2r))
    return jnp.concatenate(parts, axis=0).reshape(-1)[:P]


# speedup vs baseline: 1.3615x; 1.0014x over previous
import functools

import jax
import jax.numpy as jnp
from jax import lax
from jax.experimental import pallas as pl
from jax.experimental.pallas import tpu as pltpu
from jax.experimental.pallas import tpu_sc as plsc

N = 10000
E = 320000
P = 100000
D = 128

NPAD = 10240
PPAD = 102400
NC, NS, L = 2, 16, 16
NW = NC * NS

EPW = E // NW
ECH = 100
ENCH = EPW // ECH

PCH = 128
PPW = PPAD // NW
PNCH = PPW // PCH
PRING = 5

NB = 400
NGRID = N // NB
PB = 1024
PGRID = PPAD // PB

_mesh = plsc.VectorSubcoreMesh(core_axis_name="c", subcore_axis_name="s")


@functools.partial(
    pl.kernel,
    out_type=jax.ShapeDtypeStruct((NW, N), jnp.float32),
    mesh=_mesh,
    compiler_params=pltpu.CompilerParams(needs_layout_passes=False),
    scratch_types=[
        pltpu.VMEM((EPW,), jnp.int32),
        pltpu.VMEM((N,), jnp.float32),
    ],
)
def _deg_kernel(col_hbm, out_hbm, cidx, degv):
    w = lax.axis_index("s") * NC + lax.axis_index("c")
    pltpu.sync_copy(col_hbm.at[pl.ds(w * EPW, EPW)], cidx)
    zeros16 = jnp.zeros((L,), jnp.float32)
    ones16 = jnp.ones((L,), jnp.float32)

    def zbody(i, carry):
        degv[pl.ds(i * L, L)] = zeros16
        return carry

    lax.fori_loop(0, N // L, zbody, 0)

    def sbody(i, carry):
        idx = cidx[pl.ds(i * L, L)]
        plsc.addupdate_scatter(degv, [idx], ones16)
        return carry

    lax.fori_loop(0, EPW // L, sbody, 0)
    pltpu.sync_copy(degv, out_hbm.at[w])


@functools.partial(
    pl.kernel,
    out_type=jax.ShapeDtypeStruct((NC, NPAD, D), jnp.float32),
    mesh=_mesh,
    scratch_types=[
        [pltpu.VMEM((ECH,), jnp.int32)] * 2,
        [pltpu.VMEM((ECH,), jnp.int32)] * 2,
        [pltpu.VMEM((ECH, D), jnp.float32)] * 2,
        [pltpu.SemaphoreType.DMA] * 2,
        [pltpu.SemaphoreType.DMA] * 2,
        [pltpu.SemaphoreType.DMA] * 2,
        pltpu.VMEM_SHARED((NPAD, D), jnp.float32),
    ],
)
def _scatter_kernel(ridx_hbm, cidx_hbm, tbl_hbm, zeros_hbm, out_hbm,
                    rbuf, cbuf, rows, rsem, csem, gsem, accsh):
    c = lax.axis_index("c")
    s = lax.axis_index("s")
    w = s * NC + c

    rows_per_tile = NPAD // NS
    base = s * rows_per_tile
    pltpu.sync_copy(zeros_hbm, rows[0])
    for k in range(rows_per_tile // ECH):
        pltpu.sync_copy(rows[0], accsh.at[pl.ds(base + k * ECH, ECH)])
    rem = rows_per_tile % ECH
    pltpu.sync_copy(rows[0].at[pl.ds(0, rem)],
                    accsh.at[pl.ds(base + rows_per_tile - rem, rem)])
    plsc.subcore_barrier()

    for p in range(2):
        pltpu.sync_copy(ridx_hbm.at[w, p], rbuf[p])
        pltpu.sync_copy(cidx_hbm.at[w, p], cbuf[p])
        pltpu.async_copy(tbl_hbm.at[rbuf[p]], rows[p], gsem[p])

    def body(j, carry):
        for p in range(2):
            t = j * 2 + p
            pltpu.make_async_copy(tbl_hbm.at[rbuf[p]], rows[p], gsem[p]).wait()

            @pl.when(t + 2 < ENCH)
            def _():
                pltpu.async_copy(ridx_hbm.at[w, t + 2], rbuf[p], rsem[p])

            pltpu.sync_copy(rows[p], accsh.at[cbuf[p]], add=True)

            @pl.when(t + 2 < ENCH)
            def _():
                pltpu.async_copy(cidx_hbm.at[w, t + 2], cbuf[p], csem[p])
                pltpu.make_async_copy(ridx_hbm.at[w, t + 2], rbuf[p],
                                      rsem[p]).wait()
                pltpu.make_async_copy(cidx_hbm.at[w, t + 2], cbuf[p],
                                      csem[p]).wait()
                pltpu.async_copy(tbl_hbm.at[rbuf[p]], rows[p], gsem[p])
        return carry

    lax.fori_loop(0, ENCH // 2, body, 0)
    plsc.subcore_barrier()
    pltpu.sync_copy(accsh.at[pl.ds(base, rows_per_tile)],
                    out_hbm.at[c, pl.ds(base, rows_per_tile)])


NSL = 5
SP = PPAD // NSL
SCH0 = 2
SCH1 = 8
PRING = 4
SMAX = SCH0 + SCH1


@functools.partial(
    pl.kernel,
    out_type=jax.ShapeDtypeStruct((2, SP, D), jnp.float32),
    mesh=_mesh,
    scratch_types=[
        pltpu.VMEM((2 * SMAX, PCH), jnp.int32),
        [pltpu.VMEM((PCH, D), jnp.float32)] * PRING,
        [pltpu.SemaphoreType.DMA] * PRING,
    ],
)
def _pair_gather_kernel(z_hbm, idx_hbm, out_hbm, idx, rows, sems):
    c = lax.axis_index("c")
    s = lax.axis_index("s")
    w = s * NC + c
    pltpu.sync_copy(idx_hbm.at[w], idx)
    nch = jnp.where(c == 0, SCH0, SCH1)
    jobs = 2 * nch
    base_pair = jnp.where(c == 0, s * (SCH0 * PCH),
                          16 * (SCH0 * PCH) + s * (SCH1 * PCH))

    def _gather(t, k):
        side = t // nch
        jj = t - side * nch
        pltpu.async_copy(z_hbm.at[idx.at[side * SMAX + jj]], rows[k],
                         sems[k])

    def _drain(t, k):
        side = t // nch
        jj = t - side * nch
        pltpu.make_async_copy(z_hbm.at[idx.at[side * SMAX + jj]], rows[k],
                              sems[k]).wait()
        pltpu.sync_copy(rows[k],
                        out_hbm.at[side, pl.ds(base_pair + jj * PCH, PCH)])

    for k in range(PRING):
        _gather(k, k)

    def body(j, carry):
        for k in range(PRING):
            t = j * PRING + k
            _drain(t, k)

            @pl.when(t + PRING < jobs)
            def _():
                _gather(t + PRING, k)
        return carry

    lax.fori_loop(0, jobs // PRING, body, 0)


def _conv_pre_body(deg_ref, x_ref, w_ref, hd_ref, dinv_ref):
    dsum = jnp.sum(deg_ref[...], axis=1, keepdims=True) + 1.0
    dinv = lax.rsqrt(dsum)
    hx = jnp.dot(x_ref[...], w_ref[...], preferred_element_type=jnp.float32)
    hd_ref[...] = hx * dinv
    dinv_ref[...] = dinv


def _conv_mid_body(a_ref, hd_ref, dinv_ref, w_ref, b_ref, out_ref):
    dinv = dinv_ref[...]
    s = (a_ref[0] + a_ref[1] + hd_ref[...]) * dinv + b_ref[...]
    h = jnp.maximum(s, 0.0)
    out_ref[...] = jnp.dot(h, w_ref[...],
                           preferred_element_type=jnp.float32) * dinv


def _conv_out_body(a_ref, hd_ref, dinv_ref, b_ref, out_ref):
    out_ref[...] = (a_ref[0] + a_ref[1] + hd_ref[...]) * dinv_ref[...] \
        + b_ref[...]


def _pair_mlp_body(zsd_ref, wa_ref, wb_ref, wc_ref, wd_ref,
                   b1_ref, w2_ref, b2_ref, out_ref):
    zs = zsd_ref[0]
    zd = zsd_ref[1]
    acc = jnp.dot(zs, wa_ref[...], preferred_element_type=jnp.float32)
    acc = acc + jnp.dot(zd, wb_ref[...], preferred_element_type=jnp.float32)
    acc = acc + jnp.dot(zs * zd, wc_ref[...],
                        preferred_element_type=jnp.float32)
    acc = acc + jnp.dot(jnp.abs(zs - zd), wd_ref[...],
                        preferred_element_type=jnp.float32)
    hid = jnp.maximum(acc + b1_ref[...], 0.0)
    lv = jnp.sum(hid * w2_ref[...], axis=1, keepdims=True)
    out_ref[...] = lv + b2_ref[...]


_nblk = pl.BlockSpec((NB, D), lambda i: (i, 0))
_accblk = pl.BlockSpec((2, NB, D), lambda i: (0, i, 0))
_wblk = pl.BlockSpec((D, D), lambda i: (0, 0))
_dinvblk = pl.BlockSpec((NB, 1), lambda i: (i, 0))
_biasblk = pl.BlockSpec((1, D), lambda i: (0, 0))
_scalarblk = pl.BlockSpec((1, 1), lambda i: (0, 0))

_conv_pre = pl.pallas_call(
    _conv_pre_body,
    grid=(NGRID,),
    in_specs=[pl.BlockSpec((NB, NW), lambda i: (i, 0)), _nblk, _wblk],
    out_specs=[_nblk, _dinvblk],
    out_shape=[jax.ShapeDtypeStruct((N, D), jnp.float32),
               jax.ShapeDtypeStruct((N, 1), jnp.float32)],
)

_conv_mid = pl.pallas_call(
    _conv_mid_body,
    grid=(NGRID,),
    in_specs=[_accblk, _nblk, _dinvblk, _wblk, _biasblk],
    out_specs=_nblk,
    out_shape=jax.ShapeDtypeStruct((N, D), jnp.float32),
)

_conv_out = pl.pallas_call(
    _conv_out_body,
    grid=(NGRID,),
    in_specs=[_accblk, _nblk, _dinvblk, _biasblk],
    out_specs=_nblk,
    out_shape=jax.ShapeDtypeStruct((N, D), jnp.float32),
)

_zsdblk = pl.BlockSpec((2, PB, D), lambda i: (0, i, 0))
_wblk16 = pl.BlockSpec((D, D), lambda i: (0, 0))

_pair_mlp = pl.pallas_call(
    _pair_mlp_body,
    grid=(SP // PB,),
    in_specs=[_zsdblk, _wblk16, _wblk16, _wblk16, _wblk16,
              _biasblk, _biasblk, _scalarblk],
    out_specs=pl.BlockSpec((PB, 1), lambda i: (i, 0)),
    out_shape=jax.ShapeDtypeStruct((SP, 1), jnp.float32),
)


def kernel(x, edge_index, edge_label_index, W1, b1, W2, b2,
           Wp1, bp1, Wp2, bp2):
    f32 = jnp.float32
    row3d = edge_index[0].reshape(NW, ENCH, ECH)
    col3d = edge_index[1].reshape(NW, ENCH, ECH)
    col_flat = edge_index[1]
    eli = jnp.concatenate(
        [edge_label_index,
         jnp.zeros((2, PPAD - P), edge_label_index.dtype)], axis=1)
    zeros_blk = jnp.zeros((ECH, D), f32)

    deg32 = _deg_kernel(col_flat)
    hd1, dinv = _conv_pre(deg32.T, x, W1)
    acc1 = _scatter_kernel(row3d, col3d, hd1, zeros_blk)
    hd2 = _conv_mid(acc1, hd1, dinv, W2, b1.reshape(1, D))
    acc2 = _scatter_kernel(row3d, col3d, hd2, zeros_blk)
    z = _conv_out(acc2, hd2, dinv, b2.reshape(1, D))

    Wp = Wp1.reshape(4, D, D)
    bp1r = bp1.reshape(1, D)
    w2r = Wp2.reshape(1, D)
    bp2r = bp2.reshape(1, 1)

    def _slice_idx(v):
        c0 = v[:16 * SCH0 * PCH].reshape(16, SCH0, PCH)
        c0 = jnp.concatenate(
            [c0, jnp.zeros((16, SMAX - SCH0, PCH), v.dtype)], axis=1)
        c1 = v[16 * SCH0 * PCH:].reshape(16, SCH1, PCH)
        c1 = jnp.concatenate(
            [c1, jnp.zeros((16, SMAX - SCH1, PCH), v.dtype)], axis=1)
        return jnp.stack([c0, c1], axis=1).reshape(NW, SMAX, PCH)

    parts = []
    for sl in range(NSL):
        lo = sl * SP
        pidx_s = jnp.concatenate([_slice_idx(eli[0][lo:lo + SP]),
                                  _slice_idx(eli[1][lo:lo + SP])], axis=1)
        zsd_s = _pair_gather_kernel(z, pidx_s)
        parts.append(_pair_mlp(zsd_s, Wp[0], Wp[1], Wp[2], Wp[3],
                               bp1r, w2r, bp2r))
    return jnp.concatenate(parts, axis=0).reshape(-1)[:P]


# speedup vs baseline: 1.4127x; 1.0376x over previous
import functools

import jax
import jax.numpy as jnp
from jax import lax
from jax.experimental import pallas as pl
from jax.experimental.pallas import tpu as pltpu
from jax.experimental.pallas import tpu_sc as plsc

N = 10000
E = 320000
P = 100000
D = 128

NPAD = 10240
PPAD = 102400
NC, NS, L = 2, 16, 16
NW = NC * NS

EPW = E // NW
ECH = 100
ENCH = EPW // ECH

PCH = 128
PPW = PPAD // NW
PNCH = PPW // PCH
PRING = 5

NB = 400
NGRID = N // NB
PB = 1024
PGRID = PPAD // PB

_mesh = plsc.VectorSubcoreMesh(core_axis_name="c", subcore_axis_name="s")


@functools.partial(
    pl.kernel,
    out_type=jax.ShapeDtypeStruct((NW, N), jnp.float32),
    mesh=_mesh,
    compiler_params=pltpu.CompilerParams(needs_layout_passes=False),
    scratch_types=[
        pltpu.VMEM((EPW,), jnp.int32),
        pltpu.VMEM((N,), jnp.float32),
    ],
)
def _deg_kernel(col_hbm, out_hbm, cidx, degv):
    w = lax.axis_index("s") * NC + lax.axis_index("c")
    pltpu.sync_copy(col_hbm.at[pl.ds(w * EPW, EPW)], cidx)
    zeros16 = jnp.zeros((L,), jnp.float32)
    ones16 = jnp.ones((L,), jnp.float32)

    def zbody(i, carry):
        degv[pl.ds(i * L, L)] = zeros16
        return carry

    lax.fori_loop(0, N // L, zbody, 0)

    def sbody(i, carry):
        idx = cidx[pl.ds(i * L, L)]
        plsc.addupdate_scatter(degv, [idx], ones16)
        return carry

    lax.fori_loop(0, EPW // L, sbody, 0)
    pltpu.sync_copy(degv, out_hbm.at[w])


@functools.partial(
    pl.kernel,
    out_type=jax.ShapeDtypeStruct((NC, NPAD, D), jnp.float32),
    mesh=_mesh,
    scratch_types=[
        [pltpu.VMEM((ECH,), jnp.int32)] * 2,
        [pltpu.VMEM((ECH,), jnp.int32)] * 2,
        [pltpu.VMEM((ECH, D), jnp.float32)] * 2,
        [pltpu.SemaphoreType.DMA] * 2,
        [pltpu.SemaphoreType.DMA] * 2,
        [pltpu.SemaphoreType.DMA] * 2,
        pltpu.VMEM_SHARED((NPAD, D), jnp.float32),
    ],
)
def _scatter_kernel(ridx_hbm, cidx_hbm, tbl_hbm, zeros_hbm, out_hbm,
                    rbuf, cbuf, rows, rsem, csem, gsem, accsh):
    c = lax.axis_index("c")
    s = lax.axis_index("s")
    w = s * NC + c

    rows_per_tile = NPAD // NS
    base = s * rows_per_tile
    pltpu.sync_copy(zeros_hbm, rows[0])
    for k in range(rows_per_tile // ECH):
        pltpu.sync_copy(rows[0], accsh.at[pl.ds(base + k * ECH, ECH)])
    rem = rows_per_tile % ECH
    pltpu.sync_copy(rows[0].at[pl.ds(0, rem)],
                    accsh.at[pl.ds(base + rows_per_tile - rem, rem)])
    plsc.subcore_barrier()

    for p in range(2):
        pltpu.sync_copy(ridx_hbm.at[w, p], rbuf[p])
        pltpu.sync_copy(cidx_hbm.at[w, p], cbuf[p])
        pltpu.async_copy(tbl_hbm.at[rbuf[p]], rows[p], gsem[p])

    def body(j, carry):
        for p in range(2):
            t = j * 2 + p
            pltpu.make_async_copy(tbl_hbm.at[rbuf[p]], rows[p], gsem[p]).wait()

            @pl.when(t + 2 < ENCH)
            def _():
                pltpu.async_copy(ridx_hbm.at[w, t + 2], rbuf[p], rsem[p])

            pltpu.sync_copy(rows[p], accsh.at[cbuf[p]], add=True)

            @pl.when(t + 2 < ENCH)
            def _():
                pltpu.async_copy(cidx_hbm.at[w, t + 2], cbuf[p], csem[p])
                pltpu.make_async_copy(ridx_hbm.at[w, t + 2], rbuf[p],
                                      rsem[p]).wait()
                pltpu.make_async_copy(cidx_hbm.at[w, t + 2], cbuf[p],
                                      csem[p]).wait()
                pltpu.async_copy(tbl_hbm.at[rbuf[p]], rows[p], gsem[p])
        return carry

    lax.fori_loop(0, ENCH // 2, body, 0)
    plsc.subcore_barrier()
    pltpu.sync_copy(accsh.at[pl.ds(base, rows_per_tile)],
                    out_hbm.at[c, pl.ds(base, rows_per_tile)])


NSL = 5
SP = PPAD // NSL
SCH0 = 8
SCH1 = 2
PRING = 4
SMAX = SCH0 + SCH1


@functools.partial(
    pl.kernel,
    out_type=jax.ShapeDtypeStruct((2, SP, D), jnp.float32),
    mesh=_mesh,
    scratch_types=[
        pltpu.VMEM((2 * SMAX, PCH), jnp.int32),
        [pltpu.VMEM((PCH, D), jnp.float32)] * PRING,
        [pltpu.SemaphoreType.DMA] * PRING,
    ],
)
def _pair_gather_kernel(z_hbm, idx_hbm, out_hbm, idx, rows, sems):
    c = lax.axis_index("c")
    s = lax.axis_index("s")
    w = s * NC + c
    pltpu.sync_copy(idx_hbm.at[w], idx)
    nch = jnp.where(c == 0, SCH0, SCH1)
    jobs = 2 * nch
    base_pair = jnp.where(c == 0, s * (SCH0 * PCH),
                          16 * (SCH0 * PCH) + s * (SCH1 * PCH))

    def _gather(t, k):
        side = t // nch
        jj = t - side * nch
        pltpu.async_copy(z_hbm.at[idx.at[side * SMAX + jj]], rows[k],
                         sems[k])

    def _drain(t, k):
        side = t // nch
        jj = t - side * nch
        pltpu.make_async_copy(z_hbm.at[idx.at[side * SMAX + jj]], rows[k],
                              sems[k]).wait()
        pltpu.sync_copy(rows[k],
                        out_hbm.at[side, pl.ds(base_pair + jj * PCH, PCH)])

    for k in range(PRING):
        _gather(k, k)

    def body(j, carry):
        for k in range(PRING):
            t = j * PRING + k
            _drain(t, k)

            @pl.when(t + PRING < jobs)
            def _():
                _gather(t + PRING, k)
        return carry

    lax.fori_loop(0, jobs // PRING, body, 0)


def _conv_pre_body(deg_ref, x_ref, w_ref, hd_ref, dinv_ref):
    dsum = jnp.sum(deg_ref[...], axis=1, keepdims=True) + 1.0
    dinv = lax.rsqrt(dsum)
    hx = jnp.dot(x_ref[...], w_ref[...], preferred_element_type=jnp.float32)
    hd_ref[...] = hx * dinv
    dinv_ref[...] = dinv


def _conv_mid_body(a_ref, hd_ref, dinv_ref, w_ref, b_ref, out_ref):
    dinv = dinv_ref[...]
    s = (a_ref[0] + a_ref[1] + hd_ref[...]) * dinv + b_ref[...]
    h = jnp.maximum(s, 0.0)
    out_ref[...] = jnp.dot(h, w_ref[...],
                           preferred_element_type=jnp.float32) * dinv


def _conv_out_body(a_ref, hd_ref, dinv_ref, b_ref, out_ref):
    out_ref[...] = (a_ref[0] + a_ref[1] + hd_ref[...]) * dinv_ref[...] \
        + b_ref[...]


def _pair_mlp_body(zsd_ref, wa_ref, wb_ref, wc_ref, wd_ref,
                   b1_ref, w2_ref, b2_ref, out_ref):
    zs = zsd_ref[0]
    zd = zsd_ref[1]
    acc = jnp.dot(zs, wa_ref[...], preferred_element_type=jnp.float32)
    acc = acc + jnp.dot(zd, wb_ref[...], preferred_element_type=jnp.float32)
    acc = acc + jnp.dot(zs * zd, wc_ref[...],
                        preferred_element_type=jnp.float32)
    acc = acc + jnp.dot(jnp.abs(zs - zd), wd_ref[...],
                        preferred_element_type=jnp.float32)
    hid = jnp.maximum(acc + b1_ref[...], 0.0)
    lv = jnp.sum(hid * w2_ref[...], axis=1, keepdims=True)
    out_ref[...] = lv + b2_ref[...]


_nblk = pl.BlockSpec((NB, D), lambda i: (i, 0))
_accblk = pl.BlockSpec((2, NB, D), lambda i: (0, i, 0))
_wblk = pl.BlockSpec((D, D), lambda i: (0, 0))
_dinvblk = pl.BlockSpec((NB, 1), lambda i: (i, 0))
_biasblk = pl.BlockSpec((1, D), lambda i: (0, 0))
_scalarblk = pl.BlockSpec((1, 1), lambda i: (0, 0))

_conv_pre = pl.pallas_call(
    _conv_pre_body,
    grid=(NGRID,),
    in_specs=[pl.BlockSpec((NB, NW), lambda i: (i, 0)), _nblk, _wblk],
    out_specs=[_nblk, _dinvblk],
    out_shape=[jax.ShapeDtypeStruct((N, D), jnp.float32),
               jax.ShapeDtypeStruct((N, 1), jnp.float32)],
)

_conv_mid = pl.pallas_call(
    _conv_mid_body,
    grid=(NGRID,),
    in_specs=[_accblk, _nblk, _dinvblk, _wblk, _biasblk],
    out_specs=_nblk,
    out_shape=jax.ShapeDtypeStruct((N, D), jnp.float32),
)

_conv_out = pl.pallas_call(
    _conv_out_body,
    grid=(NGRID,),
    in_specs=[_accblk, _nblk, _dinvblk, _biasblk],
    out_specs=_nblk,
    out_shape=jax.ShapeDtypeStruct((N, D), jnp.float32),
)

_zsdblk = pl.BlockSpec((2, PB, D), lambda i: (0, i, 0))
_wblk16 = pl.BlockSpec((D, D), lambda i: (0, 0))

_pair_mlp = pl.pallas_call(
    _pair_mlp_body,
    grid=(SP // PB,),
    in_specs=[_zsdblk, _wblk16, _wblk16, _wblk16, _wblk16,
              _biasblk, _biasblk, _scalarblk],
    out_specs=pl.BlockSpec((PB, 1), lambda i: (i, 0)),
    out_shape=jax.ShapeDtypeStruct((SP, 1), jnp.float32),
)


def kernel(x, edge_index, edge_label_index, W1, b1, W2, b2,
           Wp1, bp1, Wp2, bp2):
    f32 = jnp.float32
    row3d = edge_index[0].reshape(NW, ENCH, ECH)
    col3d = edge_index[1].reshape(NW, ENCH, ECH)
    col_flat = edge_index[1]
    eli = jnp.concatenate(
        [edge_label_index,
         jnp.zeros((2, PPAD - P), edge_label_index.dtype)], axis=1)
    zeros_blk = jnp.zeros((ECH, D), f32)

    deg32 = _deg_kernel(col_flat)
    hd1, dinv = _conv_pre(deg32.T, x, W1)
    acc1 = _scatter_kernel(row3d, col3d, hd1, zeros_blk)
    hd2 = _conv_mid(acc1, hd1, dinv, W2, b1.reshape(1, D))
    acc2 = _scatter_kernel(row3d, col3d, hd2, zeros_blk)
    z = _conv_out(acc2, hd2, dinv, b2.reshape(1, D))

    Wp = Wp1.reshape(4, D, D)
    bp1r = bp1.reshape(1, D)
    w2r = Wp2.reshape(1, D)
    bp2r = bp2.reshape(1, 1)

    def _slice_idx(v):
        c0 = v[:16 * SCH0 * PCH].reshape(16, SCH0, PCH)
        c0 = jnp.concatenate(
            [c0, jnp.zeros((16, SMAX - SCH0, PCH), v.dtype)], axis=1)
        c1 = v[16 * SCH0 * PCH:].reshape(16, SCH1, PCH)
        c1 = jnp.concatenate(
            [c1, jnp.zeros((16, SMAX - SCH1, PCH), v.dtype)], axis=1)
        return jnp.stack([c0, c1], axis=1).reshape(NW, SMAX, PCH)

    parts = []
    for sl in range(NSL):
        lo = sl * SP
        pidx_s = jnp.concatenate([_slice_idx(eli[0][lo:lo + SP]),
                                  _slice_idx(eli[1][lo:lo + SP])], axis=1)
        zsd_s = _pair_gather_kernel(z, pidx_s)
        parts.append(_pair_mlp(zsd_s, Wp[0], Wp[1], Wp[2], Wp[3],
                               bp1r, w2r, bp2r))
    return jnp.concatenate(parts, axis=0).reshape(-1)[:P]
